# baseline (speedup 1.0000x reference)
import functools

import jax
import jax.numpy as jnp
import numpy as np
from jax import lax
from jax.experimental import pallas as pl
from jax.experimental.pallas import tpu as pltpu
from jax.experimental.pallas import tpu_sc as plsc

WIDTH = 16
N_NODES = 10000
N_EDGES = 320000

NC = 2
NS = 16
NW = NC * NS
E_PER_W = N_EDGES // NW
CH = 80
NCHUNK = E_PER_W // CH
GF = 5
GOUT = NCHUNK // GF
SROWS = 2000
SOUT = E_PER_W // SROWS
SIN = SROWS // CH
ROWS_PER_TILE = N_NODES // NS

_f32 = jnp.float32


GROWS = GF * CH // 8


def _sc_gather_body(h_hbm, src_hbm, out_hbm, idx_v, rows_v, gsem, osem):
    wid = lax.axis_index("s") * NC + lax.axis_index("c")
    base = wid * E_PER_W
    pltpu.sync_copy(src_hbm.at[wid], idx_v)

    def outer(o, carry):
        b = lax.rem(o, 2)

        @pl.when(o >= 2)
        def _():
            pltpu.make_async_copy(
                out_hbm.at[pl.ds(0, GF * CH)], rows_v.at[0], osem).wait()

        for g in range(GF):
            j = o * GF + g
            pltpu.async_copy(
                h_hbm.at[idx_v.at[j]], rows_v.at[b, pl.ds(g * CH, CH)], gsem)
        pltpu.make_async_copy(
            h_hbm.at[pl.ds(0, GF * CH)], rows_v.at[0], gsem).wait()
        pltpu.async_copy(
            rows_v.at[b],
            out_hbm.at[pl.ds(base + o * GF * CH, GF * CH)], osem)
        return carry

    lax.fori_loop(0, GOUT, outer, 0)
    pltpu.make_async_copy(out_hbm.at[pl.ds(0, GF * CH)], rows_v.at[0],
                          osem).wait()
    pltpu.make_async_copy(out_hbm.at[pl.ds(0, GF * CH)], rows_v.at[0],
                          osem).wait()


def _scatter_body(msg_hbm, dst_hbm, zeros_hbm, dep_hbm, outm_hbm, idx_v,
                  rows_v, aggm_sh, lsem, ssem):
    del dep_hbm
    cid = lax.axis_index("c")
    sid = lax.axis_index("s")
    wid = sid * NC + cid
    base = wid * E_PER_W
    r0 = sid * ROWS_PER_TILE
    pltpu.sync_copy(zeros_hbm.at[pl.ds(r0, ROWS_PER_TILE)],
                    aggm_sh.at[pl.ds(r0, ROWS_PER_TILE)])
    pltpu.sync_copy(dst_hbm.at[wid], idx_v)
    plsc.subcore_barrier()
    pltpu.async_copy(msg_hbm.at[pl.ds(base, SROWS)], rows_v.at[0], lsem)

    def outer(o, carry):
        b = lax.rem(o, 2)
        pltpu.make_async_copy(
            msg_hbm.at[pl.ds(0, SROWS)], rows_v.at[0], lsem).wait()

        @pl.when(o + 1 < SOUT)
        def _():
            pltpu.async_copy(
                msg_hbm.at[pl.ds(base + (o + 1) * SROWS, SROWS)],
                rows_v.at[1 - b], lsem)

        def inner(i, c2):
            j = o * SIN + i
            pltpu.async_copy(rows_v.at[b, pl.ds(i * CH, CH)],
                             aggm_sh.at[idx_v.at[j]], ssem, add=True)
            return c2

        lax.fori_loop(0, SIN, inner, 0)
        pltpu.make_async_copy(
            msg_hbm.at[pl.ds(0, SROWS)], rows_v.at[0], ssem).wait()
        return carry

    lax.fori_loop(0, SOUT, outer, 0)
    plsc.subcore_barrier()
    pltpu.sync_copy(aggm_sh.at[pl.ds(r0, ROWS_PER_TILE)],
                    outm_hbm.at[cid, pl.ds(r0, ROWS_PER_TILE)])


def _counts_body(dst_hbm, zeros_hbm, ones_hbm, dep_hbm, outc_hbm, idx_v,
                 ones_v, drain_v, aggc_sh, ssem):
    del dep_hbm
    cid = lax.axis_index("c")
    sid = lax.axis_index("s")
    wid = sid * NC + cid
    r0 = sid * ROWS_PER_TILE
    pltpu.sync_copy(zeros_hbm.at[pl.ds(r0, ROWS_PER_TILE)],
                    aggc_sh.at[pl.ds(r0, ROWS_PER_TILE)])
    pltpu.sync_copy(ones_hbm, ones_v)
    pltpu.sync_copy(dst_hbm.at[wid], idx_v)
    plsc.subcore_barrier()

    def outer(o, carry):
        def inner(i, c2):
            j = o * SIN + i
            pltpu.async_copy(ones_v, aggc_sh.at[idx_v.at[j]], ssem, add=True)
            return c2

        lax.fori_loop(0, SIN, inner, 0)
        pltpu.make_async_copy(
            zeros_hbm.at[pl.ds(0, SROWS)], drain_v, ssem).wait()
        return carry

    lax.fori_loop(0, SOUT, outer, 0)
    plsc.subcore_barrier()
    pltpu.sync_copy(aggc_sh.at[pl.ds(r0, ROWS_PER_TILE)],
                    outc_hbm.at[cid, pl.ds(r0, ROWS_PER_TILE)])


@functools.lru_cache(maxsize=1)
def _sc_kernels():
    mesh = plsc.VectorSubcoreMesh(
        core_axis_name="c", subcore_axis_name="s",
        num_cores=NC, num_subcores=NS)

    params = pltpu.CompilerParams(use_tc_tiling_on_sc=False)

    gather = functools.partial(
        pl.kernel,
        out_type=jax.ShapeDtypeStruct((N_EDGES, WIDTH), _f32),
        mesh=mesh,
        compiler_params=params,
        scratch_types=[
            pltpu.VMEM((NCHUNK, CH), jnp.int32),
            pltpu.VMEM((2, GF * CH, WIDTH), _f32),
            pltpu.SemaphoreType.DMA,
            pltpu.SemaphoreType.DMA,
        ],
    )(_sc_gather_body)

    scatter = functools.partial(
        pl.kernel,
        out_type=jax.ShapeDtypeStruct((NC, N_NODES, WIDTH), _f32),
        mesh=mesh,
        compiler_params=params,
        scratch_types=[
            pltpu.VMEM((NCHUNK, CH), jnp.int32),
            pltpu.VMEM((2, SROWS, WIDTH), _f32),
            pltpu.VMEM_SHARED((N_NODES, WIDTH), _f32),
            pltpu.SemaphoreType.DMA,
            pltpu.SemaphoreType.DMA,
        ],
    )(_scatter_body)

    counts = functools.partial(
        pl.kernel,
        out_type=jax.ShapeDtypeStruct((NC, N_NODES, WIDTH), _f32),
        mesh=mesh,
        compiler_params=params,
        scratch_types=[
            pltpu.VMEM((NCHUNK, CH), jnp.int32),
            pltpu.VMEM((CH, WIDTH), _f32),
            pltpu.VMEM((SROWS, WIDTH), _f32),
            pltpu.VMEM_SHARED((N_NODES, WIDTH), _f32),
            pltpu.SemaphoreType.DMA,
        ],
    )(_counts_body)

    return gather, scatter, counts


E8 = N_EDGES // 8
N8 = N_NODES // 8
RB = 1000

_SEL_NP = np.zeros((WIDTH, 128, 128), np.float32)
for _f in range(WIDTH):
    for _p in range(8):
        _SEL_NP[_f, _p * 16 + _f, _p * 16:_p * 16 + 16] = 1.0


def _msg8_body(a8_ref, x8_ref, selbf_ref, bdall_ref, out_ref):
    a8bf = a8_ref[...].astype(jnp.bfloat16)
    x8 = x8_ref[...]
    p_all = jnp.dot(a8bf, selbf_ref[...], preferred_element_type=_f32)
    g_all = jnp.dot(x8, bdall_ref[...], preferred_element_type=_f32)
    acc = g_all[:, WIDTH * 128:]
    for f in range(WIDTH):
        acc = acc + p_all[:, f * 128:(f + 1) * 128] * g_all[:, f * 128:(f + 1) * 128]
    out_ref[...] = acc


_msg8_call = pl.pallas_call(
    _msg8_body,
    grid=(E8 // RB,),
    in_specs=[
        pl.BlockSpec((RB, 128), lambda i: (i, 0)),
        pl.BlockSpec((RB, 128), lambda i: (i, 0)),
        pl.BlockSpec((128, WIDTH * 128), lambda i: (0, 0)),
        pl.BlockSpec((128, (WIDTH + 1) * 128), lambda i: (0, 0)),
    ],
    out_specs=pl.BlockSpec((RB, 128), lambda i: (i, 0)),
    out_shape=jax.ShapeDtypeStruct((E8, 128), _f32),
)


def _upd8_body(pm_ref, pc_ref, h_ref, wl_ref, bl_ref, out_ref):
    agg = pm_ref[0] + pm_ref[1]
    cnt = pc_ref[0] + pc_ref[1]
    mean = agg / jnp.maximum(cnt, 1.0)
    out_ref[...] = jnp.tanh(
        mean + jnp.dot(h_ref[...], wl_ref[...], preferred_element_type=_f32)
        + bl_ref[...])


_upd8_call = pl.pallas_call(
    _upd8_body,
    out_shape=jax.ShapeDtypeStruct((N8, 128), _f32),
)


def kernel(x, edge_index, edge_attr, Wd0, bd0, Wl0, bl0, Wd1, bd1, Wl1, bl1):
    src3 = edge_index[0].reshape(NW, NCHUNK, CH)
    dst3 = edge_index[1].reshape(NW, NCHUNK, CH)
    zeros_n = jnp.zeros((N_NODES, WIDTH), _f32)
    ones_ch = jnp.ones((CH, WIDTH), _f32)
    ea8 = edge_attr.reshape(E8, 128)
    selbf = jnp.asarray(_SEL_NP).transpose(1, 0, 2).reshape(
        128, WIDTH * 128).astype(jnp.bfloat16)
    eye8 = jnp.eye(8, dtype=_f32)

    def xf(Wd, bd, Wl, bl):
        w3 = Wd.reshape(WIDTH, WIDTH, WIDTH).transpose(2, 1, 0)
        bdf = jnp.einsum("pq,fji->fpjqi", eye8, w3).reshape(WIDTH, 128, 128)
        bdb = jnp.kron(eye8, bd.reshape(WIDTH, WIDTH).T)
        bdall = jnp.concatenate(
            [bdf.transpose(1, 0, 2).reshape(128, WIDTH * 128), bdb], axis=1)
        wld = jnp.kron(eye8, Wl.T)
        bl8 = jnp.tile(bl, 8).reshape(1, 128)
        return bdall, wld, bl8

    BDALL0, WLD0, BL0 = xf(Wd0, bd0, Wl0, bl0)
    BDALL1, WLD1, BL1 = xf(Wd1, bd1, Wl1, bl1)

    _sc_gather, _sc_scatter, _sc_counts = _sc_kernels()

    xj = _sc_gather(x, src3)
    pc = _sc_counts(dst3, zeros_n, ones_ch, xj)
    msg8 = _msg8_call(ea8, xj.reshape(E8, 128), selbf, BDALL0)
    pm = _sc_scatter(msg8.reshape(N_EDGES, WIDTH), dst3, zeros_n, pc)
    h8 = _upd8_call(pm.reshape(NC, N8, 128), pc.reshape(NC, N8, 128),
                    x.reshape(N8, 128), WLD0, BL0)

    xj = _sc_gather(h8.reshape(N_NODES, WIDTH), src3)
    msg8 = _msg8_call(ea8, xj.reshape(E8, 128), selbf, BDALL1)
    pm2 = _sc_scatter(msg8.reshape(N_EDGES, WIDTH), dst3, zeros_n, pc)
    h8 = _upd8_call(pm2.reshape(NC, N8, 128), pc.reshape(NC, N8, 128),
                    h8, WLD1, BL1)
    return h8.reshape(N_NODES, WIDTH)


# speedup vs baseline: 1.0511x; 1.0511x over previous
import functools

import jax
import jax.numpy as jnp
import numpy as np
from jax import lax
from jax.experimental import pallas as pl
from jax.experimental.pallas import tpu as pltpu
from jax.experimental.pallas import tpu_sc as plsc

WIDTH = 16
N_NODES = 10000
N_EDGES = 320000

NC = 2
NS = 16
NW = NC * NS
E_PER_W = N_EDGES // NW
CH = 80
NCHUNK = E_PER_W // CH
GF = 5
GOUT = NCHUNK // GF
SROWS = 2000
SOUT = E_PER_W // SROWS
SIN = SROWS // CH
ROWS_PER_TILE = N_NODES // NS

_f32 = jnp.float32


GROWS = GF * CH // 8


def _sc_gather_body(h_hbm, src_hbm, out_hbm, idx_v, rows_v, gsem, osem):
    wid = lax.axis_index("s") * NC + lax.axis_index("c")
    base = wid * E_PER_W
    h16 = h_hbm
    out16 = out_hbm
    pltpu.sync_copy(src_hbm.at[wid], idx_v)

    def outer(o, carry):
        b = lax.rem(o, 2)

        @pl.when(o >= 2)
        def _():
            pltpu.make_async_copy(
                out16.at[pl.ds(0, GF * CH)], rows_v.at[0], osem).wait()

        for g in range(GF):
            j = o * GF + g
            pltpu.async_copy(
                h16.at[idx_v.at[j]], rows_v.at[b, pl.ds(g * CH, CH)], gsem)
        pltpu.make_async_copy(
            h16.at[pl.ds(0, GF * CH)], rows_v.at[0], gsem).wait()
        pltpu.async_copy(
            rows_v.at[b], out16.at[pl.ds(base + o * GF * CH, GF * CH)], osem)
        return carry

    lax.fori_loop(0, GOUT, outer, 0)
    pltpu.make_async_copy(out16.at[pl.ds(0, GF * CH)], rows_v.at[0],
                          osem).wait()
    pltpu.make_async_copy(out16.at[pl.ds(0, GF * CH)], rows_v.at[0],
                          osem).wait()


def _scatter_body(with_cnt, msg_hbm, dst_hbm, zeros_hbm, ones_hbm,
                  outm_hbm, outc_hbm, idx_v, rows_v, ones_v, aggm_sh, aggc_sh,
                  lsem, ssem):
    cid = lax.axis_index("c")
    sid = lax.axis_index("s")
    wid = sid * NC + cid
    base = wid * E_PER_W
    r0 = sid * ROWS_PER_TILE
    msg16 = msg_hbm
    z16 = zeros_hbm
    out16 = outm_hbm
    pltpu.sync_copy(z16.at[pl.ds(r0, ROWS_PER_TILE)],
                    aggm_sh.at[pl.ds(r0, ROWS_PER_TILE)])
    if with_cnt:
        pltpu.sync_copy(z16.at[pl.ds(r0, ROWS_PER_TILE)],
                        aggc_sh.at[pl.ds(r0, ROWS_PER_TILE)])
        pltpu.sync_copy(ones_hbm, ones_v)
    pltpu.sync_copy(dst_hbm.at[wid], idx_v)
    plsc.subcore_barrier()
    pltpu.async_copy(msg16.at[pl.ds(base, SROWS)], rows_v.at[0], lsem)

    def outer(o, carry):
        b = lax.rem(o, 2)
        pltpu.make_async_copy(
            msg16.at[pl.ds(0, SROWS)], rows_v.at[0], lsem).wait()

        @pl.when(o + 1 < SOUT)
        def _():
            pltpu.async_copy(
                msg16.at[pl.ds(base + (o + 1) * SROWS, SROWS)],
                rows_v.at[1 - b], lsem)

        def inner(i, c2):
            j = o * SIN + i
            pltpu.async_copy(rows_v.at[b, pl.ds(i * CH, CH)],
                             aggm_sh.at[idx_v.at[j]], ssem, add=True)
            if with_cnt:
                pltpu.async_copy(ones_v, aggc_sh.at[idx_v.at[j]], ssem,
                                 add=True)
            return c2

        lax.fori_loop(0, SIN, inner, 0)
        pltpu.make_async_copy(
            msg16.at[pl.ds(0, SROWS)], rows_v.at[0], ssem).wait()
        if with_cnt:
            pltpu.make_async_copy(
                msg16.at[pl.ds(0, SROWS)], rows_v.at[0], ssem).wait()
        return carry

    lax.fori_loop(0, SOUT, outer, 0)
    plsc.subcore_barrier()
    pltpu.sync_copy(aggm_sh.at[pl.ds(r0, ROWS_PER_TILE)],
                    out16.at[cid, pl.ds(r0, ROWS_PER_TILE)])
    if with_cnt:
        pltpu.sync_copy(aggc_sh.at[pl.ds(r0, ROWS_PER_TILE)],
                        outc_hbm.at[cid, pl.ds(r0, ROWS_PER_TILE)])


@functools.lru_cache(maxsize=1)
def _sc_kernels():
    mesh = plsc.VectorSubcoreMesh(
        core_axis_name="c", subcore_axis_name="s",
        num_cores=NC, num_subcores=NS)

    params = pltpu.CompilerParams(use_tc_tiling_on_sc=False)

    gather = functools.partial(
        pl.kernel,
        out_type=jax.ShapeDtypeStruct((N_EDGES, WIDTH), _f32),
        mesh=mesh,
        compiler_params=params,
        scratch_types=[
            pltpu.VMEM((NCHUNK, CH), jnp.int32),
            pltpu.VMEM((2, GF * CH, WIDTH), _f32),
            pltpu.SemaphoreType.DMA,
            pltpu.SemaphoreType.DMA,
        ],
    )(_sc_gather_body)

    scatter_out2 = (jax.ShapeDtypeStruct((NC, N_NODES, WIDTH), _f32),
                    jax.ShapeDtypeStruct((NC, N_NODES, WIDTH), _f32))

    @functools.partial(pl.kernel, out_type=scatter_out2, mesh=mesh,
                       compiler_params=params,
                       scratch_types=[
                           pltpu.VMEM((NCHUNK, CH), jnp.int32),
                           pltpu.VMEM((2, SROWS, WIDTH), _f32),
                           pltpu.VMEM((CH, WIDTH), _f32),
                           pltpu.VMEM_SHARED((N_NODES, WIDTH), _f32),
                           pltpu.VMEM_SHARED((N_NODES, WIDTH), _f32),
                           pltpu.SemaphoreType.DMA,
                           pltpu.SemaphoreType.DMA,
                       ])
    def scatter_cnt(msg_hbm, dst_hbm, zeros_hbm, ones_hbm, outm_hbm, outc_hbm,
                    idx_v, rows_v, ones_v, aggm_sh, aggc_sh, lsem, ssem):
        _scatter_body(True, msg_hbm, dst_hbm, zeros_hbm, ones_hbm, outm_hbm,
                      outc_hbm, idx_v, rows_v, ones_v, aggm_sh, aggc_sh,
                      lsem, ssem)

    @functools.partial(pl.kernel,
                       out_type=jax.ShapeDtypeStruct((NC, N_NODES, WIDTH),
                                                     _f32),
                       mesh=mesh,
                       compiler_params=params,
                       scratch_types=[
                           pltpu.VMEM((NCHUNK, CH), jnp.int32),
                           pltpu.VMEM((2, SROWS, WIDTH), _f32),
                           pltpu.VMEM_SHARED((N_NODES, WIDTH), _f32),
                           pltpu.SemaphoreType.DMA,
                           pltpu.SemaphoreType.DMA,
                       ])
    def scatter(msg_hbm, dst_hbm, zeros_hbm, outm_hbm, idx_v, rows_v, aggm_sh,
                lsem, ssem):
        _scatter_body(False, msg_hbm, dst_hbm, zeros_hbm, None, outm_hbm,
                      None, idx_v, rows_v, None, aggm_sh, None, lsem, ssem)

    return gather, scatter_cnt, scatter


E8 = N_EDGES // 8
N8 = N_NODES // 8
RB = 1000

_SEL_NP = np.zeros((WIDTH, 128, 128), np.float32)
for _f in range(WIDTH):
    for _p in range(8):
        _SEL_NP[_f, _p * 16 + _f, _p * 16:_p * 16 + 16] = 1.0


def _msg8_body(a8_ref, x8_ref, selbf_ref, bdall_ref, out_ref):
    a8bf = a8_ref[...].astype(jnp.bfloat16)
    x8 = x8_ref[...]
    p_all = jnp.dot(a8bf, selbf_ref[...], preferred_element_type=_f32)
    g_all = jnp.dot(x8, bdall_ref[...], preferred_element_type=_f32)
    acc = g_all[:, WIDTH * 128:]
    for f in range(WIDTH):
        acc = acc + p_all[:, f * 128:(f + 1) * 128] * g_all[:, f * 128:(f + 1) * 128]
    out_ref[...] = acc


_msg8_call = pl.pallas_call(
    _msg8_body,
    grid=(E8 // RB,),
    in_specs=[
        pl.BlockSpec((RB, 128), lambda i: (i, 0)),
        pl.BlockSpec((RB, 128), lambda i: (i, 0)),
        pl.BlockSpec((128, WIDTH * 128), lambda i: (0, 0)),
        pl.BlockSpec((128, (WIDTH + 1) * 128), lambda i: (0, 0)),
    ],
    out_specs=pl.BlockSpec((RB, 128), lambda i: (i, 0)),
    out_shape=jax.ShapeDtypeStruct((E8, 128), _f32),
)


def _upd8_body(pm_ref, pc_ref, h_ref, wl_ref, bl_ref, out_ref):
    agg = pm_ref[0] + pm_ref[1]
    cnt = pc_ref[0] + pc_ref[1]
    mean = agg / jnp.maximum(cnt, 1.0)
    out_ref[...] = jnp.tanh(
        mean + jnp.dot(h_ref[...], wl_ref[...], preferred_element_type=_f32)
        + bl_ref[...])


_upd8_call = pl.pallas_call(
    _upd8_body,
    out_shape=jax.ShapeDtypeStruct((N8, 128), _f32),
)


def kernel(x, edge_index, edge_attr, Wd0, bd0, Wl0, bl0, Wd1, bd1, Wl1, bl1):
    src3 = edge_index[0].reshape(NW, NCHUNK, CH)
    dst3 = edge_index[1].reshape(NW, NCHUNK, CH)
    zeros_n = jnp.zeros((N_NODES, WIDTH), _f32)
    ones_ch = jnp.ones((CH, WIDTH), _f32)
    ea8 = edge_attr.reshape(E8, 128)
    selbf = jnp.asarray(_SEL_NP).transpose(1, 0, 2).reshape(
        128, WIDTH * 128).astype(jnp.bfloat16)
    eye8 = jnp.eye(8, dtype=_f32)

    def xf(Wd, bd, Wl, bl):
        w3 = Wd.reshape(WIDTH, WIDTH, WIDTH).transpose(2, 1, 0)
        bdf = jnp.einsum("pq,fji->fpjqi", eye8, w3).reshape(WIDTH, 128, 128)
        bdb = jnp.kron(eye8, bd.reshape(WIDTH, WIDTH).T)
        bdall = jnp.concatenate(
            [bdf.transpose(1, 0, 2).reshape(128, WIDTH * 128), bdb], axis=1)
        wld = jnp.kron(eye8, Wl.T)
        bl8 = jnp.tile(bl, 8).reshape(1, 128)
        return bdall, wld, bl8

    BDALL0, WLD0, BL0 = xf(Wd0, bd0, Wl0, bl0)
    BDALL1, WLD1, BL1 = xf(Wd1, bd1, Wl1, bl1)

    _sc_gather, _sc_scatter_cnt, _sc_scatter = _sc_kernels()

    xj = _sc_gather(x, src3)
    msg8 = _msg8_call(ea8, xj.reshape(E8, 128), selbf, BDALL0)
    pm, pc = _sc_scatter_cnt(msg8.reshape(N_EDGES, WIDTH), dst3, zeros_n,
                             ones_ch)
    h8 = _upd8_call(pm.reshape(NC, N8, 128), pc.reshape(NC, N8, 128),
                    x.reshape(N8, 128), WLD0, BL0)

    xj = _sc_gather(h8.reshape(N_NODES, WIDTH), src3)
    msg8 = _msg8_call(ea8, xj.reshape(E8, 128), selbf, BDALL1)
    pm2 = _sc_scatter(msg8.reshape(N_EDGES, WIDTH), dst3, zeros_n)
    h8 = _upd8_call(pm2.reshape(NC, N8, 128), pc.reshape(NC, N8, 128),
                    h8, WLD1, BL1)
    return h8.reshape(N_NODES, WIDTH)


# speedup vs baseline: 1.0711x; 1.0190x over previous
import functools

import jax
import jax.numpy as jnp
import numpy as np
from jax import lax
from jax.experimental import pallas as pl
from jax.experimental.pallas import tpu as pltpu
from jax.experimental.pallas import tpu_sc as plsc

WIDTH = 16
N_NODES = 10000
N_EDGES = 320000

NC = 2
NS = 16
NW = NC * NS
E_PER_W = N_EDGES // NW
CH = 80
NCHUNK = E_PER_W // CH
GF = 25
GOUT = NCHUNK // GF
SROWS = 2000
SOUT = E_PER_W // SROWS
SIN = SROWS // CH
ROWS_PER_TILE = N_NODES // NS

_f32 = jnp.float32


GROWS = GF * CH // 8


def _sc_gather_body(h_hbm, src_hbm, out_hbm, idx_v, rows_v, gsem, osem):
    wid = lax.axis_index("s") * NC + lax.axis_index("c")
    base = wid * E_PER_W
    h16 = h_hbm
    out16 = out_hbm
    pltpu.sync_copy(src_hbm.at[wid], idx_v)

    def outer(o, carry):
        b = lax.rem(o, 2)

        @pl.when(o >= 2)
        def _():
            pltpu.make_async_copy(
                out16.at[pl.ds(0, GF * CH)], rows_v.at[0], osem).wait()

        for g in range(GF):
            j = o * GF + g
            pltpu.async_copy(
                h16.at[idx_v.at[j]], rows_v.at[b, pl.ds(g * CH, CH)], gsem)
        pltpu.make_async_copy(
            h16.at[pl.ds(0, GF * CH)], rows_v.at[0], gsem).wait()
        pltpu.async_copy(
            rows_v.at[b], out16.at[pl.ds(base + o * GF * CH, GF * CH)], osem)
        return carry

    lax.fori_loop(0, GOUT, outer, 0)
    pltpu.make_async_copy(out16.at[pl.ds(0, GF * CH)], rows_v.at[0],
                          osem).wait()
    pltpu.make_async_copy(out16.at[pl.ds(0, GF * CH)], rows_v.at[0],
                          osem).wait()


def _scatter_body(with_cnt, msg_hbm, dst_hbm, zeros_hbm, ones_hbm,
                  outm_hbm, outc_hbm, idx_v, rows_v, ones_v, aggm_sh, aggc_sh,
                  lsem, ssem):
    cid = lax.axis_index("c")
    sid = lax.axis_index("s")
    wid = sid * NC + cid
    base = wid * E_PER_W
    r0 = sid * ROWS_PER_TILE
    msg16 = msg_hbm
    z16 = zeros_hbm
    out16 = outm_hbm
    pltpu.sync_copy(z16.at[pl.ds(r0, ROWS_PER_TILE)],
                    aggm_sh.at[pl.ds(r0, ROWS_PER_TILE)])
    if with_cnt:
        pltpu.sync_copy(z16.at[pl.ds(r0, ROWS_PER_TILE)],
                        aggc_sh.at[pl.ds(r0, ROWS_PER_TILE)])
        pltpu.sync_copy(ones_hbm, ones_v)
    pltpu.sync_copy(dst_hbm.at[wid], idx_v)
    plsc.subcore_barrier()
    pltpu.async_copy(msg16.at[pl.ds(base, SROWS)], rows_v.at[0], lsem)

    def outer(o, carry):
        b = lax.rem(o, 2)
        pltpu.make_async_copy(
            msg16.at[pl.ds(0, SROWS)], rows_v.at[0], lsem).wait()

        @pl.when(o + 1 < SOUT)
        def _():
            pltpu.async_copy(
                msg16.at[pl.ds(base + (o + 1) * SROWS, SROWS)],
                rows_v.at[1 - b], lsem)

        def inner(i, c2):
            j = o * SIN + i
            pltpu.async_copy(rows_v.at[b, pl.ds(i * CH, CH)],
                             aggm_sh.at[idx_v.at[j]], ssem, add=True)
            if with_cnt:
                pltpu.async_copy(ones_v, aggc_sh.at[idx_v.at[j]], ssem,
                                 add=True)
            return c2

        lax.fori_loop(0, SIN, inner, 0)
        pltpu.make_async_copy(
            msg16.at[pl.ds(0, SROWS)], rows_v.at[0], ssem).wait()
        if with_cnt:
            pltpu.make_async_copy(
                msg16.at[pl.ds(0, SROWS)], rows_v.at[0], ssem).wait()
        return carry

    lax.fori_loop(0, SOUT, outer, 0)
    plsc.subcore_barrier()
    pltpu.sync_copy(aggm_sh.at[pl.ds(r0, ROWS_PER_TILE)],
                    out16.at[cid, pl.ds(r0, ROWS_PER_TILE)])
    if with_cnt:
        pltpu.sync_copy(aggc_sh.at[pl.ds(r0, ROWS_PER_TILE)],
                        outc_hbm.at[cid, pl.ds(r0, ROWS_PER_TILE)])


@functools.lru_cache(maxsize=1)
def _sc_kernels():
    mesh = plsc.VectorSubcoreMesh(
        core_axis_name="c", subcore_axis_name="s",
        num_cores=NC, num_subcores=NS)

    params = pltpu.CompilerParams(use_tc_tiling_on_sc=False)

    gather = functools.partial(
        pl.kernel,
        out_type=jax.ShapeDtypeStruct((N_EDGES, WIDTH), _f32),
        mesh=mesh,
        compiler_params=params,
        scratch_types=[
            pltpu.VMEM((NCHUNK, CH), jnp.int32),
            pltpu.VMEM((2, GF * CH, WIDTH), _f32),
            pltpu.SemaphoreType.DMA,
            pltpu.SemaphoreType.DMA,
        ],
    )(_sc_gather_body)

    scatter_out2 = (jax.ShapeDtypeStruct((NC, N_NODES, WIDTH), _f32),
                    jax.ShapeDtypeStruct((NC, N_NODES, WIDTH), _f32))

    @functools.partial(pl.kernel, out_type=scatter_out2, mesh=mesh,
                       compiler_params=params,
                       scratch_types=[
                           pltpu.VMEM((NCHUNK, CH), jnp.int32),
                           pltpu.VMEM((2, SROWS, WIDTH), _f32),
                           pltpu.VMEM((CH, WIDTH), _f32),
                           pltpu.VMEM_SHARED((N_NODES, WIDTH), _f32),
                           pltpu.VMEM_SHARED((N_NODES, WIDTH), _f32),
                           pltpu.SemaphoreType.DMA,
                           pltpu.SemaphoreType.DMA,
                       ])
    def scatter_cnt(msg_hbm, dst_hbm, zeros_hbm, ones_hbm, outm_hbm, outc_hbm,
                    idx_v, rows_v, ones_v, aggm_sh, aggc_sh, lsem, ssem):
        _scatter_body(True, msg_hbm, dst_hbm, zeros_hbm, ones_hbm, outm_hbm,
                      outc_hbm, idx_v, rows_v, ones_v, aggm_sh, aggc_sh,
                      lsem, ssem)

    @functools.partial(pl.kernel,
                       out_type=jax.ShapeDtypeStruct((NC, N_NODES, WIDTH),
                                                     _f32),
                       mesh=mesh,
                       compiler_params=params,
                       scratch_types=[
                           pltpu.VMEM((NCHUNK, CH), jnp.int32),
                           pltpu.VMEM((2, SROWS, WIDTH), _f32),
                           pltpu.VMEM_SHARED((N_NODES, WIDTH), _f32),
                           pltpu.SemaphoreType.DMA,
                           pltpu.SemaphoreType.DMA,
                       ])
    def scatter(msg_hbm, dst_hbm, zeros_hbm, outm_hbm, idx_v, rows_v, aggm_sh,
                lsem, ssem):
        _scatter_body(False, msg_hbm, dst_hbm, zeros_hbm, None, outm_hbm,
                      None, idx_v, rows_v, None, aggm_sh, None, lsem, ssem)

    return gather, scatter_cnt, scatter


E8 = N_EDGES // 8
N8 = N_NODES // 8
RB = 1000

_SEL_NP = np.zeros((WIDTH, 128, 128), np.float32)
for _f in range(WIDTH):
    for _p in range(8):
        _SEL_NP[_f, _p * 16 + _f, _p * 16:_p * 16 + 16] = 1.0


def _msg8_body(a8_ref, x8_ref, selbf_ref, bdall_ref, out_ref):
    a8bf = a8_ref[...].astype(jnp.bfloat16)
    x8 = x8_ref[...]
    p_all = jnp.dot(a8bf, selbf_ref[...], preferred_element_type=_f32)
    g_all = jnp.dot(x8, bdall_ref[...], preferred_element_type=_f32)
    acc = g_all[:, WIDTH * 128:]
    for f in range(WIDTH):
        acc = acc + p_all[:, f * 128:(f + 1) * 128] * g_all[:, f * 128:(f + 1) * 128]
    out_ref[...] = acc


_msg8_call = pl.pallas_call(
    _msg8_body,
    grid=(E8 // RB,),
    in_specs=[
        pl.BlockSpec((RB, 128), lambda i: (i, 0)),
        pl.BlockSpec((RB, 128), lambda i: (i, 0)),
        pl.BlockSpec((128, WIDTH * 128), lambda i: (0, 0)),
        pl.BlockSpec((128, (WIDTH + 1) * 128), lambda i: (0, 0)),
    ],
    out_specs=pl.BlockSpec((RB, 128), lambda i: (i, 0)),
    out_shape=jax.ShapeDtypeStruct((E8, 128), _f32),
)


def _upd8_body(pm_ref, pc_ref, h_ref, wl_ref, bl_ref, out_ref):
    agg = pm_ref[0] + pm_ref[1]
    cnt = pc_ref[0] + pc_ref[1]
    mean = agg / jnp.maximum(cnt, 1.0)
    out_ref[...] = jnp.tanh(
        mean + jnp.dot(h_ref[...], wl_ref[...], preferred_element_type=_f32)
        + bl_ref[...])


_upd8_call = pl.pallas_call(
    _upd8_body,
    out_shape=jax.ShapeDtypeStruct((N8, 128), _f32),
)


def kernel(x, edge_index, edge_attr, Wd0, bd0, Wl0, bl0, Wd1, bd1, Wl1, bl1):
    src3 = edge_index[0].reshape(NW, NCHUNK, CH)
    dst3 = edge_index[1].reshape(NW, NCHUNK, CH)
    zeros_n = jnp.zeros((N_NODES, WIDTH), _f32)
    ones_ch = jnp.ones((CH, WIDTH), _f32)
    ea8 = edge_attr.reshape(E8, 128)
    selbf = jnp.asarray(_SEL_NP).transpose(1, 0, 2).reshape(
        128, WIDTH * 128).astype(jnp.bfloat16)
    eye8 = jnp.eye(8, dtype=_f32)

    def xf(Wd, bd, Wl, bl):
        w3 = Wd.reshape(WIDTH, WIDTH, WIDTH).transpose(2, 1, 0)
        bdf = jnp.einsum("pq,fji->fpjqi", eye8, w3).reshape(WIDTH, 128, 128)
        bdb = jnp.kron(eye8, bd.reshape(WIDTH, WIDTH).T)
        bdall = jnp.concatenate(
            [bdf.transpose(1, 0, 2).reshape(128, WIDTH * 128), bdb], axis=1)
        wld = jnp.kron(eye8, Wl.T)
        bl8 = jnp.tile(bl, 8).reshape(1, 128)
        return bdall, wld, bl8

    BDALL0, WLD0, BL0 = xf(Wd0, bd0, Wl0, bl0)
    BDALL1, WLD1, BL1 = xf(Wd1, bd1, Wl1, bl1)

    _sc_gather, _sc_scatter_cnt, _sc_scatter = _sc_kernels()

    xj = _sc_gather(x, src3)
    msg8 = _msg8_call(ea8, xj.reshape(E8, 128), selbf, BDALL0)
    pm, pc = _sc_scatter_cnt(msg8.reshape(N_EDGES, WIDTH), dst3, zeros_n,
                             ones_ch)
    h8 = _upd8_call(pm.reshape(NC, N8, 128), pc.reshape(NC, N8, 128),
                    x.reshape(N8, 128), WLD0, BL0)

    xj = _sc_gather(h8.reshape(N_NODES, WIDTH), src3)
    msg8 = _msg8_call(ea8, xj.reshape(E8, 128), selbf, BDALL1)
    pm2 = _sc_scatter(msg8.reshape(N_EDGES, WIDTH), dst3, zeros_n)
    h8 = _upd8_call(pm2.reshape(NC, N8, 128), pc.reshape(NC, N8, 128),
                    h8, WLD1, BL1)
    return h8.reshape(N_NODES, WIDTH)


# speedup vs baseline: 1.0738x; 1.0025x over previous
import functools

import jax
import jax.numpy as jnp
import numpy as np
from jax import lax
from jax.experimental import pallas as pl
from jax.experimental.pallas import tpu as pltpu
from jax.experimental.pallas import tpu_sc as plsc

WIDTH = 16
N_NODES = 10000
N_EDGES = 320000

NC = 2
NS = 16
NW = NC * NS
E_PER_W = N_EDGES // NW
CH = 80
NCHUNK = E_PER_W // CH
GF = 25
GOUT = NCHUNK // GF
SROWS = 2000
SOUT = E_PER_W // SROWS
SIN = SROWS // CH
ROWS_PER_TILE = N_NODES // NS

_f32 = jnp.float32


def _sc_gather_body(h_hbm, src_hbm, out_hbm, idx_v, rows_v, gsem, osem):
    wid = lax.axis_index("s") * NC + lax.axis_index("c")
    base = wid * E_PER_W
    h16 = h_hbm
    out16 = out_hbm
    pltpu.sync_copy(src_hbm.at[wid], idx_v)

    def outer(o, carry):
        b = lax.rem(o, 2)

        @pl.when(o >= 2)
        def _():
            pltpu.make_async_copy(
                out16.at[pl.ds(0, GF * CH)], rows_v.at[0], osem).wait()

        for g in range(GF):
            j = o * GF + g
            pltpu.async_copy(
                h16.at[idx_v.at[j]], rows_v.at[b, pl.ds(g * CH, CH)], gsem)
        pltpu.make_async_copy(
            h16.at[pl.ds(0, GF * CH)], rows_v.at[0], gsem).wait()
        pltpu.async_copy(
            rows_v.at[b], out16.at[pl.ds(base + o * GF * CH, GF * CH)], osem)
        return carry

    lax.fori_loop(0, GOUT, outer, 0)
    pltpu.make_async_copy(out16.at[pl.ds(0, GF * CH)], rows_v.at[0],
                          osem).wait()
    pltpu.make_async_copy(out16.at[pl.ds(0, GF * CH)], rows_v.at[0],
                          osem).wait()


def _scatter_body(with_cnt, msg_hbm, dst_hbm, zeros_hbm, ones_hbm,
                  outm_hbm, outc_hbm, idx_v, rows_v, ones_v, aggm_sh, aggc_sh,
                  lsem, ssem):
    cid = lax.axis_index("c")
    sid = lax.axis_index("s")
    wid = sid * NC + cid
    base = wid * E_PER_W
    r0 = sid * ROWS_PER_TILE
    msg16 = msg_hbm
    z16 = zeros_hbm
    out16 = outm_hbm
    pltpu.sync_copy(z16.at[pl.ds(r0, ROWS_PER_TILE)],
                    aggm_sh.at[pl.ds(r0, ROWS_PER_TILE)])
    if with_cnt:
        pltpu.sync_copy(z16.at[pl.ds(r0, ROWS_PER_TILE)],
                        aggc_sh.at[pl.ds(r0, ROWS_PER_TILE)])
        pltpu.sync_copy(ones_hbm, ones_v)
    pltpu.sync_copy(dst_hbm.at[wid], idx_v)
    plsc.subcore_barrier()
    pltpu.async_copy(msg16.at[pl.ds(base, SROWS)], rows_v.at[0], lsem)

    def outer(o, carry):
        b = lax.rem(o, 2)
        pltpu.make_async_copy(
            msg16.at[pl.ds(0, SROWS)], rows_v.at[0], lsem).wait()

        @pl.when(o + 1 < SOUT)
        def _():
            pltpu.async_copy(
                msg16.at[pl.ds(base + (o + 1) * SROWS, SROWS)],
                rows_v.at[1 - b], lsem)

        def inner(i, c2):
            j = o * SIN + i
            pltpu.async_copy(rows_v.at[b, pl.ds(i * CH, CH)],
                             aggm_sh.at[idx_v.at[j]], ssem, add=True)
            if with_cnt:
                pltpu.async_copy(ones_v, aggc_sh.at[idx_v.at[j]], ssem,
                                 add=True)
            return c2

        lax.fori_loop(0, SIN, inner, 0)
        pltpu.make_async_copy(
            msg16.at[pl.ds(0, SROWS)], rows_v.at[0], ssem).wait()
        if with_cnt:
            pltpu.make_async_copy(
                msg16.at[pl.ds(0, SROWS)], rows_v.at[0], ssem).wait()
        return carry

    lax.fori_loop(0, SOUT, outer, 0)
    plsc.subcore_barrier()
    pltpu.sync_copy(aggm_sh.at[pl.ds(r0, ROWS_PER_TILE)],
                    out16.at[cid, pl.ds(r0, ROWS_PER_TILE)])
    if with_cnt:
        pltpu.sync_copy(aggc_sh.at[pl.ds(r0, ROWS_PER_TILE)],
                        outc_hbm.at[cid, pl.ds(r0, ROWS_PER_TILE)])


@functools.lru_cache(maxsize=1)
def _sc_kernels():
    mesh = plsc.VectorSubcoreMesh(
        core_axis_name="c", subcore_axis_name="s",
        num_cores=NC, num_subcores=NS)

    params = pltpu.CompilerParams(use_tc_tiling_on_sc=False)

    gather = functools.partial(
        pl.kernel,
        out_type=jax.ShapeDtypeStruct((N_EDGES, WIDTH), _f32),
        mesh=mesh,
        compiler_params=params,
        scratch_types=[
            pltpu.VMEM((NCHUNK, CH), jnp.int32),
            pltpu.VMEM((2, GF * CH, WIDTH), _f32),
            pltpu.SemaphoreType.DMA,
            pltpu.SemaphoreType.DMA,
        ],
    )(_sc_gather_body)

    scatter_out2 = (jax.ShapeDtypeStruct((NC, N_NODES, WIDTH), _f32),
                    jax.ShapeDtypeStruct((NC, N_NODES, WIDTH), _f32))

    @functools.partial(pl.kernel, out_type=scatter_out2, mesh=mesh,
                       compiler_params=params,
                       scratch_types=[
                           pltpu.VMEM((NCHUNK, CH), jnp.int32),
                           pltpu.VMEM((2, SROWS, WIDTH), _f32),
                           pltpu.VMEM((CH, WIDTH), _f32),
                           pltpu.VMEM_SHARED((N_NODES, WIDTH), _f32),
                           pltpu.VMEM_SHARED((N_NODES, WIDTH), _f32),
                           pltpu.SemaphoreType.DMA,
                           pltpu.SemaphoreType.DMA,
                       ])
    def scatter_cnt(msg_hbm, dst_hbm, zeros_hbm, ones_hbm, outm_hbm, outc_hbm,
                    idx_v, rows_v, ones_v, aggm_sh, aggc_sh, lsem, ssem):
        _scatter_body(True, msg_hbm, dst_hbm, zeros_hbm, ones_hbm, outm_hbm,
                      outc_hbm, idx_v, rows_v, ones_v, aggm_sh, aggc_sh,
                      lsem, ssem)

    @functools.partial(pl.kernel,
                       out_type=jax.ShapeDtypeStruct((NC, N_NODES, WIDTH),
                                                     _f32),
                       mesh=mesh,
                       compiler_params=params,
                       scratch_types=[
                           pltpu.VMEM((NCHUNK, CH), jnp.int32),
                           pltpu.VMEM((2, SROWS, WIDTH), _f32),
                           pltpu.VMEM_SHARED((N_NODES, WIDTH), _f32),
                           pltpu.SemaphoreType.DMA,
                           pltpu.SemaphoreType.DMA,
                       ])
    def scatter(msg_hbm, dst_hbm, zeros_hbm, outm_hbm, idx_v, rows_v, aggm_sh,
                lsem, ssem):
        _scatter_body(False, msg_hbm, dst_hbm, zeros_hbm, None, outm_hbm,
                      None, idx_v, rows_v, None, aggm_sh, None, lsem, ssem)

    return gather, scatter_cnt, scatter


E8 = N_EDGES // 8
N8 = N_NODES // 8
RB = 1000

_SEL_NP = np.zeros((WIDTH, 128, 128), np.float32)
for _f in range(WIDTH):
    for _p in range(8):
        _SEL_NP[_f, _p * 16 + _f, _p * 16:_p * 16 + 16] = 1.0


def _msg8_body(a8_ref, x8_ref, selbf_ref, bdall_ref, out_ref):
    a8bf = a8_ref[...].astype(jnp.bfloat16)
    x8 = x8_ref[...]
    p_all = jnp.dot(a8bf, selbf_ref[...], preferred_element_type=_f32)
    g_all = jnp.dot(x8, bdall_ref[...], preferred_element_type=_f32)
    acc = g_all[:, WIDTH * 128:]
    for f in range(WIDTH):
        acc = acc + p_all[:, f * 128:(f + 1) * 128] * g_all[:, f * 128:(f + 1) * 128]
    out_ref[...] = acc


_msg8_call = pl.pallas_call(
    _msg8_body,
    grid=(E8 // RB,),
    in_specs=[
        pl.BlockSpec((RB, 128), lambda i: (i, 0)),
        pl.BlockSpec((RB, 128), lambda i: (i, 0)),
        pl.BlockSpec((128, WIDTH * 128), lambda i: (0, 0)),
        pl.BlockSpec((128, (WIDTH + 1) * 128), lambda i: (0, 0)),
    ],
    out_specs=pl.BlockSpec((RB, 128), lambda i: (i, 0)),
    out_shape=jax.ShapeDtypeStruct((E8, 128), _f32),
)


def _upd8_body(pm_ref, pc_ref, h_ref, wl_ref, bl_ref, out_ref):
    agg = pm_ref[0] + pm_ref[1]
    cnt = pc_ref[0] + pc_ref[1]
    mean = agg / jnp.maximum(cnt, 1.0)
    out_ref[...] = jnp.tanh(
        mean + jnp.dot(h_ref[...], wl_ref[...], preferred_element_type=_f32)
        + bl_ref[...])


_upd8_call = pl.pallas_call(
    _upd8_body,
    out_shape=jax.ShapeDtypeStruct((N8, 128), _f32),
)


def kernel(x, edge_index, edge_attr, Wd0, bd0, Wl0, bl0, Wd1, bd1, Wl1, bl1):
    src3 = edge_index[0].reshape(NW, NCHUNK, CH)
    dst3 = edge_index[1].reshape(NW, NCHUNK, CH)
    zeros_n = jnp.zeros((N_NODES, WIDTH), _f32)
    ones_ch = jnp.ones((CH, WIDTH), _f32)
    ea8 = edge_attr.reshape(E8, 128)
    selbf = jnp.asarray(_SEL_NP).transpose(1, 0, 2).reshape(
        128, WIDTH * 128).astype(jnp.bfloat16)
    eye8 = jnp.eye(8, dtype=_f32)

    def xf(Wd, bd, Wl, bl):
        w3 = Wd.reshape(WIDTH, WIDTH, WIDTH).transpose(2, 1, 0)
        bdf = jnp.einsum("pq,fji->fpjqi", eye8, w3).reshape(WIDTH, 128, 128)
        bdb = jnp.kron(eye8, bd.reshape(WIDTH, WIDTH).T)
        bdall = jnp.concatenate(
            [bdf.transpose(1, 0, 2).reshape(128, WIDTH * 128), bdb], axis=1)
        wld = jnp.kron(eye8, Wl.T)
        bl8 = jnp.tile(bl, 8).reshape(1, 128)
        return bdall, wld, bl8

    BDALL0, WLD0, BL0 = xf(Wd0, bd0, Wl0, bl0)
    BDALL1, WLD1, BL1 = xf(Wd1, bd1, Wl1, bl1)

    _sc_gather, _sc_scatter_cnt, _sc_scatter = _sc_kernels()

    xj = _sc_gather(x, src3)
    msg8 = _msg8_call(ea8, xj.reshape(E8, 128), selbf, BDALL0)
    pm, pc = _sc_scatter_cnt(msg8.reshape(N_EDGES, WIDTH), dst3, zeros_n,
                             ones_ch)
    h8 = _upd8_call(pm.reshape(NC, N8, 128), pc.reshape(NC, N8, 128),
                    x.reshape(N8, 128), WLD0, BL0)

    xj = _sc_gather(h8.reshape(N_NODES, WIDTH), src3)
    msg8 = _msg8_call(ea8, xj.reshape(E8, 128), selbf, BDALL1)
    pm2 = _sc_scatter(msg8.reshape(N_EDGES, WIDTH), dst3, zeros_n)
    h8 = _upd8_call(pm2.reshape(NC, N8, 128), pc.reshape(NC, N8, 128),
                    h8, WLD1, BL1)
    return h8.reshape(N_NODES, WIDTH)
